# baseline (speedup 1.0000x reference)
import functools

import jax
import jax.numpy as jnp
from jax import lax
from jax.experimental import pallas as pl
from jax.experimental.pallas import tpu as pltpu
from jax.experimental.pallas import tpu_sc as plsc

_SEQ = 4096
_HEAD = 256


@jax.jit
def _rope_gather(cos_tab, sin_tab, idx):
    info = plsc.get_sparse_core_info()
    nw = info.num_cores * info.num_subcores
    b_per_w = _SEQ // nw
    mesh = plsc.VectorSubcoreMesh(core_axis_name="c", subcore_axis_name="s")

    nch = 1
    rows = b_per_w // nch
    half = _HEAD // 2

    @functools.partial(
        pl.kernel,
        mesh=mesh,
        out_type=[
            jax.ShapeDtypeStruct((_SEQ, _HEAD), jnp.float32),
            jax.ShapeDtypeStruct((_SEQ, _HEAD), jnp.float32),
        ],
        scratch_types=[
            pltpu.VMEM((b_per_w,), jnp.int32),
            pltpu.VMEM((nch, rows, _HEAD), jnp.float32),
            pltpu.VMEM((nch, rows, _HEAD), jnp.float32),
        ]
        + [pltpu.SemaphoreType.DMA] * (nch + 1),
    )
    def k(cos_hbm, sin_hbm, idx_hbm, cos_out, sin_out, idx_v,
          cos_v, sin_v, *sems):
        sem_g, sem_o = sems[:nch], sems[nch]
        wid = lax.axis_index("s") * info.num_cores + lax.axis_index("c")
        base = wid * b_per_w
        pltpu.sync_copy(idx_hbm.at[0, pl.ds(base, b_per_w)], idx_v)
        gathers = []
        for c in range(nch):
            idx_c = idx_v.at[pl.ds(c * rows, rows)]
            gathers.append((
                pltpu.async_copy(cos_hbm.at[idx_c], cos_v.at[c], sem_g[c]),
                pltpu.async_copy(sin_hbm.at[idx_c], sin_v.at[c], sem_g[c]),
            ))
        outs = []
        for c in range(nch):
            gathers[c][0].wait()
            gathers[c][1].wait()
            r = pl.ds(base + c * rows, rows)
            outs.append(pltpu.async_copy(cos_v.at[c], cos_out.at[r], sem_o))
            outs.append(pltpu.async_copy(sin_v.at[c], sin_out.at[r], sem_o))
        for o in outs:
            o.wait()

    return k(cos_tab, sin_tab, idx)


def kernel(x, position_ids, cos_cached, sin_cached):
    cos, sin = _rope_gather(cos_cached[0], sin_cached[0],
                            position_ids.astype(jnp.int32))
    return cos[None].astype(x.dtype), sin[None].astype(x.dtype)


# speedup vs baseline: 1.0019x; 1.0019x over previous
import functools

import jax
import jax.numpy as jnp
from jax import lax
from jax.experimental import pallas as pl
from jax.experimental.pallas import tpu as pltpu
from jax.experimental.pallas import tpu_sc as plsc

_SEQ = 4096
_HEAD = 256


@jax.jit
def _rope_gather(cos_tab, sin_tab, idx):
    info = plsc.get_sparse_core_info()
    nw = info.num_cores * info.num_subcores
    b_per_w = _SEQ // nw
    mesh = plsc.VectorSubcoreMesh(core_axis_name="c", subcore_axis_name="s")

    nch = 1
    rows = b_per_w // nch

    @functools.partial(
        pl.kernel,
        mesh=mesh,
        out_type=[
            jax.ShapeDtypeStruct((_SEQ, _HEAD), jnp.float32),
            jax.ShapeDtypeStruct((_SEQ, _HEAD), jnp.float32),
        ],
        scratch_types=[
            pltpu.VMEM((b_per_w,), jnp.int32),
            pltpu.VMEM((nch, rows, _HEAD), jnp.float32),
            pltpu.VMEM((nch, rows, _HEAD), jnp.float32),
        ]
        + [pltpu.SemaphoreType.DMA] * (nch + 1),
    )
    def k(cos_hbm, sin_hbm, idx_hbm, cos_out, sin_out, idx_v,
          cos_v, sin_v, *sems):
        sem_g, sem_o = sems[:nch], sems[nch]
        wid = lax.axis_index("s") * info.num_cores + lax.axis_index("c")
        base = wid * b_per_w
        pltpu.sync_copy(idx_hbm.at[0, pl.ds(base, b_per_w)], idx_v)
        gathers = []
        for c in range(nch):
            idx_c = idx_v.at[pl.ds(c * rows, rows)]
            gathers.append((
                pltpu.async_copy(cos_hbm.at[idx_c], cos_v.at[c], sem_g[c]),
                pltpu.async_copy(sin_hbm.at[idx_c], sin_v.at[c], sem_g[c]),
            ))
        outs = []
        for c in range(nch):
            gathers[c][0].wait()
            gathers[c][1].wait()
            r = pl.ds(base + c * rows, rows)
            outs.append(pltpu.async_copy(cos_v.at[c], cos_out.at[r], sem_o))
            outs.append(pltpu.async_copy(sin_v.at[c], sin_out.at[r], sem_o))
        for o in outs:
            o.wait()

    return k(cos_tab, sin_tab, idx)


def kernel(x, position_ids, cos_cached, sin_cached):
    cos, sin = _rope_gather(cos_cached[0], sin_cached[0],
                            position_ids.astype(jnp.int32))
    return cos[None].astype(x.dtype), sin[None].astype(x.dtype)
